# baseline (speedup 1.0000x reference)
import functools

import jax
import jax.numpy as jnp
from jax.experimental import pallas as pl


def _conv_stats_kernel(x_ref, w_ref, h_ref, s_ref, *, rb, n_s, co):
    b = pl.program_id(0)
    r = pl.program_id(1)

    @pl.when((b == 0) & (r == 0))
    def _init():
        s_ref[...] = jnp.zeros_like(s_ref)

    acc = jnp.zeros((rb * n_s, co), jnp.float32)
    for ky in range(3):
        base = (r * rb + ky) * n_s
        acc += jnp.dot(x_ref[0, pl.ds(base, rb * n_s), :], w_ref[ky],
                       preferred_element_type=jnp.float32)
    s_ref[...] += jnp.concatenate(
        [jnp.sum(acc, axis=0, keepdims=True),
         jnp.sum(acc * acc, axis=0, keepdims=True)], axis=0)
    h_ref[0] = acc.astype(jnp.bfloat16)


def _bn_proj_kernel(h_ref, s_ref, gb_ref, w2_ref, b2_ref, o_ref, *, eps):
    mean = s_ref[0, 0:1, :]
    var = s_ref[0, 1:2, :] - mean * mean
    rstd = jax.lax.rsqrt(var + eps)
    scale = (gb_ref[0, 0:1, :] * rstd).astype(jnp.bfloat16)
    shift = (gb_ref[0, 1:2, :] - mean * gb_ref[0, 0:1, :] * rstd)
    shift = shift.astype(jnp.bfloat16)
    z = h_ref[...] * scale + shift
    y = jnp.maximum(z, jnp.bfloat16(0.1) * z)
    o_ref[...] = (jnp.dot(y, w2_ref[0], preferred_element_type=jnp.float32)
                  + b2_ref[0])


def _pass1(x, p, rb):
    B, ic, S, _ = x.shape
    co = p['w1'].shape[0]

    xp = jnp.pad(jnp.transpose(x.astype(jnp.bfloat16), (0, 2, 3, 1)),
                 ((0, 0), (1, 1), (1, 1), (0, 0)))
    xf = jnp.concatenate([xp[:, :, k:k + S, :] for k in range(3)],
                         axis=3).reshape(B, (S + 2) * S, 3 * ic)
    w1t = jnp.transpose(p['w1'].astype(jnp.bfloat16),
                        (2, 3, 1, 0)).reshape(3, 3 * ic, co)

    nrb = S // rb
    return pl.pallas_call(
        functools.partial(_conv_stats_kernel, rb=rb, n_s=S, co=co),
        grid=(B, nrb),
        in_specs=[
            pl.BlockSpec((1, (S + 2) * S, 3 * ic), lambda b, r: (b, 0, 0)),
            pl.BlockSpec((3, 3 * ic, co), lambda b, r: (0, 0, 0)),
        ],
        out_specs=[
            pl.BlockSpec((1, rb * S, co), lambda b, r: (b, r, 0)),
            pl.BlockSpec((2, co), lambda b, r: (0, 0)),
        ],
        out_shape=[
            jax.ShapeDtypeStruct((B, S * S, co), jnp.bfloat16),
            jax.ShapeDtypeStruct((2, co), jnp.float32),
        ],
    )(xf, w1t)


_SCALE_CFG = ((32,), (32,), (16,))
_MB = 1024


def _pass2_all(h1s, statss, params, n_totals):
    co = h1s[0].shape[-1]
    no = params[0]['w2'].shape[0]
    nop = ((no + 127) // 128) * 128

    ms = [h.shape[0] * h.shape[1] for h in h1s]
    h1f = jnp.concatenate([h.reshape(-1, co) for h in h1s], axis=0)
    sn = jnp.stack([s / n for s, n in zip(statss, n_totals)], axis=0)
    gb = jnp.stack([jnp.stack([p['g'], p['b']], axis=0) for p in params],
                   axis=0).astype(jnp.float32)
    w2t = jnp.stack([jnp.pad(p['w2'].reshape(no, co).T,
                             ((0, 0), (0, nop - no))) for p in params],
                    axis=0).astype(jnp.bfloat16)
    b2p = jnp.stack([jnp.pad(p['b2'], (0, nop - no)).reshape(1, nop)
                     for p in params], axis=0).astype(jnp.float32)

    M = sum(ms)
    nb = [m // _MB for m in ms]
    c0, c1 = nb[0], nb[0] + nb[1]

    def _sel(i):
        return (i >= c0).astype(jnp.int32) + (i >= c1).astype(jnp.int32)

    out = pl.pallas_call(
        functools.partial(_bn_proj_kernel, eps=1e-5),
        grid=(M // _MB,),
        in_specs=[
            pl.BlockSpec((_MB, co), lambda i: (i, 0)),
            pl.BlockSpec((1, 2, co), lambda i: (_sel(i), 0, 0)),
            pl.BlockSpec((1, 2, co), lambda i: (_sel(i), 0, 0)),
            pl.BlockSpec((1, co, nop), lambda i: (_sel(i), 0, 0)),
            pl.BlockSpec((1, 1, nop), lambda i: (_sel(i), 0, 0)),
        ],
        out_specs=pl.BlockSpec((_MB, nop), lambda i: (i, 0)),
        out_shape=jax.ShapeDtypeStruct((M, nop), jnp.float32),
    )(h1f, sn, gb, w2t, b2p)
    return out, ms, no, nop


def kernel(feat0, feat1, feat2, params):
    feats = (feat0, feat1, feat2)
    p1 = [_pass1(x, p, rb)
          for x, p, (rb,) in zip(feats, params, _SCALE_CFG)]
    n_totals = [x.shape[0] * x.shape[2] * x.shape[3] for x in feats]
    out, ms, no, nop = _pass2_all([h for h, _ in p1], [s for _, s in p1],
                                  params, n_totals)
    outs, off = [], 0
    for x, m in zip(feats, ms):
        B, _, S, _ = x.shape
        outs.append(out[off:off + m].reshape(B, S, S, nop)[..., :no])
        off += m
    return tuple(outs)


# speedup vs baseline: 1.0004x; 1.0004x over previous
import functools

import jax
import jax.numpy as jnp
from jax.experimental import pallas as pl


def _conv_stats_kernel(x_ref, w_ref, h_ref, s_ref, *, rb, n_s, co):
    b = pl.program_id(0)
    r = pl.program_id(1)

    @pl.when((b == 0) & (r == 0))
    def _init():
        s_ref[...] = jnp.zeros_like(s_ref)

    acc = jnp.zeros((rb * n_s, co), jnp.float32)
    for ky in range(3):
        base = (r * rb + ky) * n_s
        acc += jnp.dot(x_ref[0, pl.ds(base, rb * n_s), :], w_ref[ky],
                       preferred_element_type=jnp.float32)
    s_ref[...] += jnp.concatenate(
        [jnp.sum(acc, axis=0, keepdims=True),
         jnp.sum(acc * acc, axis=0, keepdims=True)], axis=0)
    h_ref[0] = acc.astype(jnp.bfloat16)


def _bn_proj_kernel(h_ref, s_ref, gb_ref, w2_ref, b2_ref, o_ref, *, eps):
    mean = s_ref[0, 0:1, :]
    var = s_ref[0, 1:2, :] - mean * mean
    rstd = jax.lax.rsqrt(var + eps)
    scale = (gb_ref[0, 0:1, :] * rstd).astype(jnp.bfloat16)
    shift = (gb_ref[0, 1:2, :] - mean * gb_ref[0, 0:1, :] * rstd)
    shift = shift.astype(jnp.bfloat16)
    z = h_ref[...] * scale + shift
    y = jnp.maximum(z, jnp.bfloat16(0.1) * z)
    o_ref[...] = (jnp.dot(y, w2_ref[0], preferred_element_type=jnp.float32)
                  + b2_ref[0])


def _pass1(x, p, rb):
    B, ic, S, _ = x.shape
    co = p['w1'].shape[0]

    xp = jnp.pad(jnp.transpose(x, (0, 2, 3, 1)),
                 ((0, 0), (1, 1), (1, 1), (0, 0))).astype(jnp.bfloat16)
    xf = jnp.concatenate([xp[:, :, k:k + S, :] for k in range(3)],
                         axis=3).reshape(B, (S + 2) * S, 3 * ic)
    w1t = jnp.transpose(p['w1'], (2, 3, 1, 0)).reshape(3, 3 * ic, co)
    w1t = w1t.astype(jnp.bfloat16)

    nrb = S // rb
    return pl.pallas_call(
        functools.partial(_conv_stats_kernel, rb=rb, n_s=S, co=co),
        grid=(B, nrb),
        in_specs=[
            pl.BlockSpec((1, (S + 2) * S, 3 * ic), lambda b, r: (b, 0, 0)),
            pl.BlockSpec((3, 3 * ic, co), lambda b, r: (0, 0, 0)),
        ],
        out_specs=[
            pl.BlockSpec((1, rb * S, co), lambda b, r: (b, r, 0)),
            pl.BlockSpec((2, co), lambda b, r: (0, 0)),
        ],
        out_shape=[
            jax.ShapeDtypeStruct((B, S * S, co), jnp.bfloat16),
            jax.ShapeDtypeStruct((2, co), jnp.float32),
        ],
    )(xf, w1t)


_SCALE_CFG = ((32,), (32,), (16,))
_MB = 1024


def _pass2_all(h1s, statss, params, n_totals):
    co = h1s[0].shape[-1]
    no = params[0]['w2'].shape[0]
    nop = ((no + 127) // 128) * 128

    ms = [h.shape[0] * h.shape[1] for h in h1s]
    h1f = jnp.concatenate([h.reshape(-1, co) for h in h1s], axis=0)
    sn = jnp.stack([s / n for s, n in zip(statss, n_totals)], axis=0)
    gb = jnp.stack([jnp.stack([p['g'], p['b']], axis=0) for p in params],
                   axis=0).astype(jnp.float32)
    w2t = jnp.stack([jnp.pad(p['w2'].reshape(no, co).T,
                             ((0, 0), (0, nop - no))) for p in params],
                    axis=0).astype(jnp.bfloat16)
    b2p = jnp.stack([jnp.pad(p['b2'], (0, nop - no)).reshape(1, nop)
                     for p in params], axis=0).astype(jnp.float32)

    M = sum(ms)
    nb = [m // _MB for m in ms]
    c0, c1 = nb[0], nb[0] + nb[1]

    def _sel(i):
        return (i >= c0).astype(jnp.int32) + (i >= c1).astype(jnp.int32)

    out = pl.pallas_call(
        functools.partial(_bn_proj_kernel, eps=1e-5),
        grid=(M // _MB,),
        in_specs=[
            pl.BlockSpec((_MB, co), lambda i: (i, 0)),
            pl.BlockSpec((1, 2, co), lambda i: (_sel(i), 0, 0)),
            pl.BlockSpec((1, 2, co), lambda i: (_sel(i), 0, 0)),
            pl.BlockSpec((1, co, nop), lambda i: (_sel(i), 0, 0)),
            pl.BlockSpec((1, 1, nop), lambda i: (_sel(i), 0, 0)),
        ],
        out_specs=pl.BlockSpec((_MB, nop), lambda i: (i, 0)),
        out_shape=jax.ShapeDtypeStruct((M, nop), jnp.float32),
    )(h1f, sn, gb, w2t, b2p)
    return out, ms, no, nop


def kernel(feat0, feat1, feat2, params):
    feats = (feat0, feat1, feat2)
    p1 = [_pass1(x, p, rb)
          for x, p, (rb,) in zip(feats, params, _SCALE_CFG)]
    n_totals = [x.shape[0] * x.shape[2] * x.shape[3] for x in feats]
    out, ms, no, nop = _pass2_all([h for h, _ in p1], [s for _, s in p1],
                                  params, n_totals)
    outs, off = [], 0
    for x, m in zip(feats, ms):
        B, _, S, _ = x.shape
        outs.append(out[off:off + m].reshape(B, S, S, nop)[..., :no])
        off += m
    return tuple(outs)
